# baseline (speedup 1.0000x reference)
import functools

import jax
import jax.numpy as jnp
from jax import lax
from jax.experimental import pallas as pl
from jax.experimental.pallas import tpu as pltpu
from jax.experimental.pallas import tpu_sc as plsc

B, N, C = 8, 20000, 91
S_PAD = 2048
NC, NS, L = 2, 16, 16
NW = NC * NS
CHUNK = S_PAD // NW
OUT_COLS = 7


def _sc_gather(boxes_flat, scores_flat, bidx, lidx, nidx):
    mesh = plsc.VectorSubcoreMesh(core_axis_name="c", subcore_axis_name="s")

    @functools.partial(
        pl.kernel,
        mesh=mesh,
        out_type=jax.ShapeDtypeStruct((OUT_COLS * S_PAD,), jnp.float32),
        scratch_types=[
            pltpu.VMEM((CHUNK,), jnp.int32),
            pltpu.VMEM((CHUNK,), jnp.int32),
            pltpu.VMEM((CHUNK,), jnp.int32),
            pltpu.VMEM((4, CHUNK), jnp.int32),
            pltpu.VMEM((CHUNK,), jnp.int32),
            pltpu.VMEM((4, CHUNK), jnp.float32),
            pltpu.VMEM((CHUNK,), jnp.float32),
            pltpu.VMEM((CHUNK,), jnp.float32),
            pltpu.VMEM((CHUNK,), jnp.float32),
            pltpu.SemaphoreType.DMA,
        ],
    )
    def k(boxes_hbm, scores_hbm, bidx_hbm, lidx_hbm, nidx_hbm, out_hbm,
          b_v, l_v, n_v, bcidx_v, elem_v, bc_v, score_v, bf_v, lf_v, sem):
        wid = lax.axis_index("s") * NC + lax.axis_index("c")
        base = wid * CHUNK

        pltpu.sync_copy(bidx_hbm.at[pl.ds(base, CHUNK)], b_v)
        pltpu.sync_copy(lidx_hbm.at[pl.ds(base, CHUNK)], l_v)
        pltpu.sync_copy(nidx_hbm.at[pl.ds(base, CHUNK)], n_v)

        for j in range(CHUNK // L):
            sl = pl.ds(j * L, L)
            row = b_v[sl] * N + n_v[sl]
            row4 = row * 4
            for c in range(4):
                bcidx_v[c, sl] = row4 + c
            elem_v[sl] = row * C + l_v[sl]
            bf_v[sl] = b_v[sl].astype(jnp.float32)
            lf_v[sl] = l_v[sl].astype(jnp.float32)

        copies = []
        for c in range(4):
            copies.append(pltpu.async_copy(
                boxes_hbm.at[bcidx_v.at[c]], bc_v.at[c], sem))
        copies.append(pltpu.async_copy(scores_hbm.at[elem_v], score_v, sem))
        for cp in copies:
            cp.wait()

        pltpu.sync_copy(bf_v, out_hbm.at[pl.ds(base, CHUNK)])
        for c in range(4):
            pltpu.sync_copy(
                bc_v.at[c],
                out_hbm.at[pl.ds((1 + c) * S_PAD + base, CHUNK)])
        pltpu.sync_copy(score_v, out_hbm.at[pl.ds(5 * S_PAD + base, CHUNK)])
        pltpu.sync_copy(lf_v, out_hbm.at[pl.ds(6 * S_PAD + base, CHUNK)])

    return k(boxes_flat, scores_flat, bidx, lidx, nidx)


def kernel(pred_boxes, pred_scores, selected_indexes):
    S = selected_indexes.shape[0]
    sel = selected_indexes.astype(jnp.int32)
    sel = jnp.pad(sel, ((0, S_PAD - S), (0, 0)))
    boxes_flat = pred_boxes.reshape(B * N * 4)
    scores_flat = pred_scores.reshape(B * N * C)
    out = _sc_gather(boxes_flat, scores_flat,
                     sel[:, 0], sel[:, 1], sel[:, 2])
    return out.reshape(OUT_COLS, S_PAD).T[:S]
